# baseline (speedup 1.0000x reference)
import functools

import jax
import jax.numpy as jnp
from jax import lax
from jax.experimental import pallas as pl
from jax.experimental.pallas import tpu as pltpu
from jax.experimental.pallas import tpu_sc as plsc

B = 4096
N_TOK = 200
D = 64
V = 1000000
NC = 2
NS = 16
NW = NC * NS
BL = 128
NBC = B // BL
E8 = D // 8
TP = 2
NP = N_TOK // TP

_mesh = plsc.VectorSubcoreMesh(core_axis_name="c", subcore_axis_name="s")


@functools.partial(
    pl.kernel,
    mesh=_mesh,
    compiler_params=pltpu.CompilerParams(
        use_tc_tiling_on_sc=True, needs_layout_passes=False
    ),
    out_type=jax.ShapeDtypeStruct((N_TOK, E8, NBC, 8, BL), jnp.float32),
    scratch_types=[
        pltpu.VMEM((2, TP, BL), jnp.int32),
        pltpu.VMEM((2, TP, BL), jnp.int32),
        pltpu.VMEM((2, TP, BL), jnp.int32),
        pltpu.VMEM((2, TP, BL, 2 * D), jnp.float32),
        pltpu.VMEM((2, TP, E8, 1, 8, BL), jnp.float32),
        pltpu.VMEM((N_TOK * D,), jnp.float32),
        pltpu.SemaphoreType.DMA((2,)),
        pltpu.SemaphoreType.DMA((2,)),
        pltpu.SemaphoreType.DMA((2,)),
    ],
)
def _emb_kernel(xp_hbm, table_hbm, pos_hbm, out_hbm, idx_v, gidx_v, par_v,
                rows_v, stage_v, pos_v, isem, gsem, wsem):
    bc = lax.axis_index("s") * NC + lax.axis_index("c")
    pltpu.sync_copy(pos_hbm, pos_v)
    viota = lax.iota(jnp.int32, 16)

    def idx_src(p):
        return xp_hbm.at[p // 4, bc, pl.ds((p % 4) * 2, TP)]

    def split_idx(sbuf):
        for j in range(TP):
            for r0 in range(0, BL, 16):
                sl = pl.ds(r0, 16)
                xv = idx_v[sbuf, j, sl]
                gidx_v[sbuf, j, sl] = xv >> 1
                par_v[sbuf, j, sl] = (xv & 1) << 6

    def launch_gathers(sbuf):
        for j in range(TP):
            pltpu.async_copy(
                table_hbm.at[gidx_v.at[sbuf, j]], rows_v.at[sbuf, j],
                gsem.at[sbuf],
            )

    pltpu.sync_copy(idx_src(0), idx_v.at[0])
    split_idx(0)
    launch_gathers(0)
    pltpu.async_copy(idx_src(1), idx_v.at[1], isem.at[1])

    def pair_body(p, carry):
        buf = lax.rem(p, 2)
        nbuf = 1 - buf
        t = p * TP

        for j in range(TP):
            pltpu.make_async_copy(
                table_hbm.at[gidx_v.at[buf, j]], rows_v.at[buf, j],
                gsem.at[buf],
            ).wait()

        @pl.when(p + 1 < NP)
        def _():
            pltpu.make_async_copy(
                idx_src(p + 1), idx_v.at[nbuf], isem.at[nbuf]
            ).wait()
            split_idx(nbuf)
            launch_gathers(nbuf)

        @pl.when(p + 2 < NP)
        def _():
            pltpu.async_copy(idx_src(p + 2), idx_v.at[buf], isem.at[buf])

        @pl.when(p >= 2)
        def _():
            pltpu.make_async_copy(
                stage_v.at[buf],
                out_hbm.at[pl.ds(t, TP), :, pl.ds(bc, 1)],
                wsem.at[buf],
            ).wait()

        @plsc.parallel_loop(0, TP * D, 1, unroll=2)
        def _ek_body(i):
            j = i >> 6
            ii = i & 63
            rot = (viota + ii) & 15
            e0 = ii & 48
            cvec = rot + e0
            e8vec = cvec >> 3
            esvec = cvec & 7
            p_vec = plsc.load_gather(pos_v, [rot + ((t + j) * D + e0)])
            rows_j = rows_v.at[buf, j]
            stage_j = stage_v.at[buf, j]
            for r0 in range(0, BL, 16):
                rvec = viota + r0
                par = par_v[buf, j, pl.ds(r0, 16)]
                vals = plsc.load_gather(rows_j, [rvec, cvec + par])
                plsc.store_scatter(
                    stage_j, [e8vec, jnp.zeros((16,), jnp.int32), esvec, rvec],
                    vals + p_vec,
                )

        pltpu.async_copy(
            stage_v.at[buf],
            out_hbm.at[pl.ds(t, TP), :, pl.ds(bc, 1)],
            wsem.at[buf],
        )
        return carry

    lax.fori_loop(0, NP, pair_body, 0)

    for k in range(2):
        pltpu.make_async_copy(
            stage_v.at[k],
            out_hbm.at[pl.ds((NP - 2 + k) * TP, TP), :, pl.ds(bc, 1)],
            wsem.at[k],
        ).wait()


def kernel(x, table, pos_embedding):
    xp = (
        x.astype(jnp.int32)
        .T.reshape(N_TOK // 8, 8, NBC, BL)
        .transpose((0, 2, 1, 3))
    )
    table2 = table.reshape(V // 2, 2 * D)
    y = _emb_kernel(xp, table2, pos_embedding.reshape(-1))
    return y.transpose((2, 4, 0, 1, 3)).reshape(B, N_TOK, D)


# speedup vs baseline: 1.7585x; 1.7585x over previous
import functools

import jax
import jax.numpy as jnp
from jax import lax
from jax.experimental import pallas as pl
from jax.experimental.pallas import tpu as pltpu
from jax.experimental.pallas import tpu_sc as plsc

B = 4096
N_TOK = 200
D = 64
V = 1000000
NC = 2
NS = 16
NW = NC * NS
BL = 128
NBC = B // BL
E8 = D // 8
TP = 2
NP = N_TOK // TP

VT = (V + BL - 1) // BL
VT_FULL = V // BL
VP_PAD = VT * (BL // 2)
A_ITERS = (VT + NW - 1) // NW

_mesh = plsc.VectorSubcoreMesh(core_axis_name="c", subcore_axis_name="s")
_cparams = pltpu.CompilerParams(
    use_tc_tiling_on_sc=True, needs_layout_passes=False
)


@functools.partial(
    pl.kernel,
    mesh=_mesh,
    compiler_params=_cparams,
    out_type=jax.ShapeDtypeStruct((VP_PAD, 2 * D), jnp.float32),
    scratch_types=[
        pltpu.VMEM((2, D, BL), jnp.float32),
        pltpu.VMEM((2, D, BL), jnp.float32),
        pltpu.SemaphoreType.DMA((2,)),
        pltpu.SemaphoreType.DMA((2,)),
    ],
)
def _fmt_kernel(tt_hbm, out_hbm, s_v, stage_v, gsem, wsem):
    wid = lax.axis_index("s") * NC + lax.axis_index("c")
    viota = lax.iota(jnp.int32, 16)

    def issue_read(vt, sbuf):
        @pl.when(vt <= VT_FULL)
        def _():
            pltpu.async_copy(
                tt_hbm.at[:, pl.ds(vt * BL, BL)], s_v.at[sbuf], gsem.at[sbuf]
            )

    issue_read(wid, 0)

    def tile_body(i, carry):
        vt = wid + i * NW
        buf = lax.rem(i, 2)
        nbuf = 1 - buf

        issue_read(wid + (i + 1) * NW, nbuf)

        @pl.when(vt <= VT_FULL)
        def _():
            pltpu.make_async_copy(
                tt_hbm.at[:, pl.ds(vt * BL, BL)], s_v.at[buf], gsem.at[buf]
            ).wait()

        @pl.when(jnp.logical_and(vt <= VT_FULL, i >= 2))
        def _():
            pltpu.make_async_copy(
                stage_v.at[buf],
                out_hbm.at[pl.ds(vt * (BL // 2), D), :],
                wsem.at[buf],
            ).wait()

        @pl.when(vt <= VT_FULL)
        def _():
            s_buf = s_v.at[buf]
            st_buf = stage_v.at[buf]

            @plsc.parallel_loop(0, D, 1, unroll=2)
            def _ek_body(q):
                rot = (viota + q) & 15
                evec = rot + (q & 48)
                for vl0 in range(0, BL, 16):
                    vlvec = viota + vl0
                    vals = plsc.load_gather(s_buf, [evec, vlvec])
                    wvec = vlvec >> 1
                    cvec = ((vlvec & 1) << 6) | evec
                    plsc.store_scatter(st_buf, [wvec, cvec], vals)

            pltpu.async_copy(
                stage_v.at[buf],
                out_hbm.at[pl.ds(vt * (BL // 2), D), :],
                wsem.at[buf],
            )

        return carry

    lax.fori_loop(0, A_ITERS, tile_body, 0)

    last_i = (VT - 1 - wid) // NW

    for k in range(2):
        li = last_i - k

        @pl.when(li >= 0)
        def _():
            pltpu.make_async_copy(
                stage_v.at[lax.rem(li, 2)],
                out_hbm.at[pl.ds((wid + li * NW) * (BL // 2), D), :],
                wsem.at[lax.rem(li, 2)],
            ).wait()


@functools.partial(
    pl.kernel,
    mesh=_mesh,
    compiler_params=_cparams,
    out_type=jax.ShapeDtypeStruct((N_TOK, E8, NBC, 8, BL), jnp.float32),
    scratch_types=[
        pltpu.VMEM((2, TP, BL), jnp.int32),
        pltpu.VMEM((2, TP, BL), jnp.int32),
        pltpu.VMEM((2, TP, BL), jnp.int32),
        pltpu.VMEM((2, TP, BL, 2 * D), jnp.float32),
        pltpu.VMEM((2, TP, E8, 1, 8, BL), jnp.float32),
        pltpu.VMEM((N_TOK * D,), jnp.float32),
        pltpu.SemaphoreType.DMA((2,)),
        pltpu.SemaphoreType.DMA((2,)),
        pltpu.SemaphoreType.DMA((2,)),
    ],
)
def _emb_kernel(xp_hbm, table_hbm, pos_hbm, out_hbm, idx_v, gidx_v, par_v,
                rows_v, stage_v, pos_v, isem, gsem, wsem):
    bc = lax.axis_index("s") * NC + lax.axis_index("c")
    pltpu.sync_copy(pos_hbm, pos_v)
    viota = lax.iota(jnp.int32, 16)

    def idx_src(p):
        return xp_hbm.at[p // 4, bc, pl.ds((p % 4) * 2, TP)]

    def split_idx(sbuf):
        for j in range(TP):
            for r0 in range(0, BL, 16):
                sl = pl.ds(r0, 16)
                xv = idx_v[sbuf, j, sl]
                gidx_v[sbuf, j, sl] = xv >> 1
                par_v[sbuf, j, sl] = (xv & 1) << 6

    def launch_gathers(sbuf):
        for j in range(TP):
            pltpu.async_copy(
                table_hbm.at[gidx_v.at[sbuf, j]], rows_v.at[sbuf, j],
                gsem.at[sbuf],
            )

    pltpu.sync_copy(idx_src(0), idx_v.at[0])
    split_idx(0)
    launch_gathers(0)
    pltpu.async_copy(idx_src(1), idx_v.at[1], isem.at[1])

    def pair_body(p, carry):
        buf = lax.rem(p, 2)
        nbuf = 1 - buf
        t = p * TP

        for j in range(TP):
            pltpu.make_async_copy(
                table_hbm.at[gidx_v.at[buf, j]], rows_v.at[buf, j],
                gsem.at[buf],
            ).wait()

        @pl.when(p + 1 < NP)
        def _():
            pltpu.make_async_copy(
                idx_src(p + 1), idx_v.at[nbuf], isem.at[nbuf]
            ).wait()
            split_idx(nbuf)
            launch_gathers(nbuf)

        @pl.when(p + 2 < NP)
        def _():
            pltpu.async_copy(idx_src(p + 2), idx_v.at[buf], isem.at[buf])

        @pl.when(p >= 2)
        def _():
            pltpu.make_async_copy(
                stage_v.at[buf],
                out_hbm.at[pl.ds(t, TP), :, pl.ds(bc, 1)],
                wsem.at[buf],
            ).wait()

        @plsc.parallel_loop(0, TP * D, 1, unroll=2)
        def _ek_body(i):
            j = i >> 6
            ii = i & 63
            rot = (viota + ii) & 15
            e0 = ii & 48
            cvec = rot + e0
            e8vec = cvec >> 3
            esvec = cvec & 7
            p_vec = plsc.load_gather(pos_v, [rot + ((t + j) * D + e0)])
            rows_j = rows_v.at[buf, j]
            stage_j = stage_v.at[buf, j]
            for r0 in range(0, BL, 16):
                rvec = viota + r0
                par = par_v[buf, j, pl.ds(r0, 16)]
                vals = plsc.load_gather(rows_j, [rvec, cvec + par])
                plsc.store_scatter(
                    stage_j, [e8vec, jnp.zeros((16,), jnp.int32), esvec, rvec],
                    vals + p_vec,
                )

        pltpu.async_copy(
            stage_v.at[buf],
            out_hbm.at[pl.ds(t, TP), :, pl.ds(bc, 1)],
            wsem.at[buf],
        )
        return carry

    lax.fori_loop(0, NP, pair_body, 0)

    for k in range(2):
        pltpu.make_async_copy(
            stage_v.at[k],
            out_hbm.at[pl.ds((NP - 2 + k) * TP, TP), :, pl.ds(bc, 1)],
            wsem.at[k],
        ).wait()


def kernel(x, table, pos_embedding):
    xp = (
        x.astype(jnp.int32)
        .T.reshape(N_TOK // 8, 8, NBC, BL)
        .transpose((0, 2, 1, 3))
    )
    table2 = _fmt_kernel(table.T)
    y = _emb_kernel(xp, table2, pos_embedding.reshape(-1))
    return y.transpose((2, 4, 0, 1, 3)).reshape(B, N_TOK, D)


# speedup vs baseline: 2.1371x; 1.2153x over previous
import functools

import jax
import jax.numpy as jnp
from jax import lax
from jax.experimental import pallas as pl
from jax.experimental.pallas import tpu as pltpu
from jax.experimental.pallas import tpu_sc as plsc

B = 4096
N_TOK = 200
D = 64
V = 1000000
NC = 2
NS = 16
NW = NC * NS
BL = 128
NBC = B // BL
E8 = D // 8
TP = 2
NP = N_TOK // TP

VT = (V + BL - 1) // BL
VT_FULL = V // BL
VP_PAD = VT * (BL // 2)
A_ITERS = (VT + NW - 1) // NW

_mesh = plsc.VectorSubcoreMesh(core_axis_name="c", subcore_axis_name="s")
_cparams = pltpu.CompilerParams(
    use_tc_tiling_on_sc=True, needs_layout_passes=False
)


@functools.partial(
    pl.kernel,
    mesh=_mesh,
    compiler_params=_cparams,
    out_type=jax.ShapeDtypeStruct((VP_PAD, 2 * D), jnp.float32),
    scratch_types=[
        pltpu.VMEM((2, D, BL), jnp.float32),
        pltpu.VMEM((2, D, BL), jnp.float32),
        pltpu.SemaphoreType.DMA((2,)),
        pltpu.SemaphoreType.DMA((2,)),
    ],
)
def _fmt_kernel(tt_hbm, out_hbm, s_v, stage_v, gsem, wsem):
    wid = lax.axis_index("s") * NC + lax.axis_index("c")
    viota = lax.iota(jnp.int32, 16)

    def issue_read(vt, sbuf):
        @pl.when(vt <= VT_FULL)
        def _():
            pltpu.async_copy(
                tt_hbm.at[:, pl.ds(vt * BL, BL)], s_v.at[sbuf], gsem.at[sbuf]
            )

    issue_read(wid, 0)

    def tile_body(i, carry):
        vt = wid + i * NW
        buf = lax.rem(i, 2)
        nbuf = 1 - buf

        issue_read(wid + (i + 1) * NW, nbuf)

        @pl.when(vt <= VT_FULL)
        def _():
            pltpu.make_async_copy(
                tt_hbm.at[:, pl.ds(vt * BL, BL)], s_v.at[buf], gsem.at[buf]
            ).wait()

        @pl.when(jnp.logical_and(vt <= VT_FULL, i >= 2))
        def _():
            pltpu.make_async_copy(
                stage_v.at[buf],
                out_hbm.at[pl.ds(vt * (BL // 2), D), :],
                wsem.at[buf],
            ).wait()

        @pl.when(vt <= VT_FULL)
        def _():
            s_buf = s_v.at[buf]
            st_buf = stage_v.at[buf]

            @plsc.parallel_loop(0, D, 1, unroll=2)
            def _ek_body(q):
                rot = (viota + q) & 15
                evec = rot + (q & 48)
                for vl0 in range(0, BL, 16):
                    vlvec = viota + vl0
                    vals = plsc.load_gather(s_buf, [evec, vlvec])
                    wvec = vlvec >> 1
                    cvec = ((vlvec & 1) << 6) | evec
                    plsc.store_scatter(st_buf, [wvec, cvec], vals)

            pltpu.async_copy(
                stage_v.at[buf],
                out_hbm.at[pl.ds(vt * (BL // 2), D), :],
                wsem.at[buf],
            )

        return carry

    lax.fori_loop(0, A_ITERS, tile_body, 0)

    last_i = (VT - 1 - wid) // NW

    for k in range(2):
        li = last_i - k

        @pl.when(li >= 0)
        def _():
            pltpu.make_async_copy(
                stage_v.at[lax.rem(li, 2)],
                out_hbm.at[pl.ds((wid + li * NW) * (BL // 2), D), :],
                wsem.at[lax.rem(li, 2)],
            ).wait()


@functools.partial(
    pl.kernel,
    mesh=_mesh,
    compiler_params=pltpu.CompilerParams(
        use_tc_tiling_on_sc=False, needs_layout_passes=False
    ),
    out_type=jax.ShapeDtypeStruct((N_TOK, E8, NBC, 8, BL), jnp.float32),
    scratch_types=[
        pltpu.VMEM((2, TP, BL), jnp.int32),
        pltpu.VMEM((2, TP, BL, D), jnp.float32),
        pltpu.VMEM((2, TP, E8, 1, 8, BL), jnp.float32),
        pltpu.VMEM((N_TOK * D,), jnp.float32),
        pltpu.SemaphoreType.DMA((2,)),
        pltpu.SemaphoreType.DMA((2,)),
        pltpu.SemaphoreType.DMA((2,)),
    ],
)
def _emb_kernel(xp_hbm, table_hbm, pos_hbm, out_hbm, idx_v, rows_v, stage_v,
                pos_v, isem, gsem, wsem):
    bc = lax.axis_index("s") * NC + lax.axis_index("c")
    pltpu.sync_copy(pos_hbm, pos_v)
    viota = lax.iota(jnp.int32, 16)

    def idx_src(p):
        return xp_hbm.at[p // 4, bc, pl.ds((p % 4) * 2, TP)]

    def launch_gathers(sbuf):
        for j in range(TP):
            pltpu.async_copy(
                table_hbm.at[idx_v.at[sbuf, j]], rows_v.at[sbuf, j],
                gsem.at[sbuf],
            )

    pltpu.sync_copy(idx_src(0), idx_v.at[0])
    launch_gathers(0)
    pltpu.async_copy(idx_src(1), idx_v.at[1], isem.at[1])

    def pair_body(p, carry):
        buf = lax.rem(p, 2)
        nbuf = 1 - buf
        t = p * TP

        for j in range(TP):
            pltpu.make_async_copy(
                table_hbm.at[idx_v.at[buf, j]], rows_v.at[buf, j],
                gsem.at[buf],
            ).wait()

        @pl.when(p + 1 < NP)
        def _():
            pltpu.make_async_copy(
                idx_src(p + 1), idx_v.at[nbuf], isem.at[nbuf]
            ).wait()
            launch_gathers(nbuf)

        @pl.when(p + 2 < NP)
        def _():
            pltpu.async_copy(idx_src(p + 2), idx_v.at[buf], isem.at[buf])

        @pl.when(p >= 2)
        def _():
            pltpu.make_async_copy(
                stage_v.at[buf],
                out_hbm.at[pl.ds(t, TP), :, pl.ds(bc, 1)],
                wsem.at[buf],
            ).wait()

        @plsc.parallel_loop(0, TP * D, 1, unroll=2)
        def _ek_body(i):
            j = i >> 6
            ii = i & 63
            rot = (viota + ii) & 15
            e0 = ii & 48
            cvec = rot + e0
            e8vec = cvec >> 3
            esvec = cvec & 7
            p_vec = plsc.load_gather(pos_v, [rot + ((t + j) * D + e0)])
            rows_j = rows_v.at[buf, j]
            stage_j = stage_v.at[buf, j]
            for r0 in range(0, BL, 16):
                rvec = viota + r0
                vals = plsc.load_gather(rows_j, [rvec, cvec])
                plsc.store_scatter(
                    stage_j, [e8vec, jnp.zeros((16,), jnp.int32), esvec, rvec],
                    vals + p_vec,
                )

        pltpu.async_copy(
            stage_v.at[buf],
            out_hbm.at[pl.ds(t, TP), :, pl.ds(bc, 1)],
            wsem.at[buf],
        )
        return carry

    lax.fori_loop(0, NP, pair_body, 0)

    for k in range(2):
        pltpu.make_async_copy(
            stage_v.at[k],
            out_hbm.at[pl.ds((NP - 2 + k) * TP, TP), :, pl.ds(bc, 1)],
            wsem.at[k],
        ).wait()


def kernel(x, table, pos_embedding):
    xp = (
        x.astype(jnp.int32)
        .T.reshape(N_TOK // 8, 8, NBC, BL)
        .transpose((0, 2, 1, 3))
    )
    table2 = _fmt_kernel(table.T)
    tlin = table2.reshape(2 * VP_PAD, D)
    y = _emb_kernel(xp, tlin, pos_embedding.reshape(-1))
    return y.transpose((2, 4, 0, 1, 3)).reshape(B, N_TOK, D)
